# baseline (speedup 1.0000x reference)
import functools

import jax
import jax.numpy as jnp
from jax.experimental import pallas as pl
from jax.experimental.pallas import tpu as pltpu

N = 10000
BI = 200
BX = 2000


def _s1_body(x_ref, w1_ref, s1_ref):
    s1_ref[...] = jnp.dot(x_ref[...], w1_ref[...],
                          preferred_element_type=jnp.float32)


def _gcn_body(adja_ref, adjb_ref, s1_ref, b1_ref, w2_ref, b2_ref,
              out_ref, s2_ref):
    p = pl.program_id(0)
    i = pl.program_id(1)

    @pl.when(p == 0)
    def _layer1():
        for k, a_ref in enumerate((adja_ref, adjb_ref)):
            h = jnp.dot(a_ref[...], s1_ref[...],
                        preferred_element_type=jnp.float32) + b1_ref[...]
            h = jnp.maximum(h, 0.0)
            s2_ref[pl.ds((2 * i + k) * BI, BI), :] = jnp.dot(
                h, w2_ref[...], preferred_element_type=jnp.float32)

    @pl.when(p == 1)
    def _layer2():
        out_ref[:BI, :] = jnp.dot(adja_ref[...], s2_ref[...],
                                  preferred_element_type=jnp.float32) + b2_ref[...]
        out_ref[BI:, :] = jnp.dot(adjb_ref[...], s2_ref[...],
                                  preferred_element_type=jnp.float32) + b2_ref[...]


@functools.partial(jax.jit, static_argnames=("interpret",))
def _gcn(x, adj, W1, b1, W2, b2, interpret=False):
    nfeat = x.shape[1]
    nhid = W1.shape[1]
    nclass = W2.shape[1]

    s1 = pl.pallas_call(
        _s1_body,
        grid=(N // BX,),
        in_specs=[
            pl.BlockSpec((BX, nfeat), lambda i: (i, 0)),
            pl.BlockSpec((nfeat, nhid), lambda i: (0, 0)),
        ],
        out_specs=pl.BlockSpec((BX, nhid), lambda i: (i, 0)),
        out_shape=jax.ShapeDtypeStruct((N, nhid), jnp.float32),
        interpret=interpret,
    )(x, W1)

    return pl.pallas_call(
        _gcn_body,
        grid=(2, N // (2 * BI)),
        in_specs=[
            pl.BlockSpec((BI, N), lambda p, i: (2 * i, 0)),
            pl.BlockSpec((BI, N), lambda p, i: (2 * i + 1, 0)),
            pl.BlockSpec((N, nhid), lambda p, i: (0, 0)),
            pl.BlockSpec((1, nhid), lambda p, i: (0, 0)),
            pl.BlockSpec((nhid, nclass), lambda p, i: (0, 0)),
            pl.BlockSpec((1, nclass), lambda p, i: (0, 0)),
        ],
        out_specs=pl.BlockSpec(
            (2 * BI, nclass), lambda p, i: (jnp.where(p == 1, i, 0), 0)),
        out_shape=jax.ShapeDtypeStruct((N, nclass), jnp.float32),
        scratch_shapes=[
            pltpu.VMEM((N, nclass), jnp.float32),
        ],
        interpret=interpret,
    )(adj, adj, s1, b1.reshape(1, -1), W2, b2.reshape(1, -1))


def kernel(x, adj, W1, b1, W2, b2):
    return _gcn(x, adj, W1, b1, W2, b2)
